# Initial kernel scaffold; baseline (speedup 1.0000x reference)
import jax
import jax.numpy as jnp
from jax.experimental import pallas as pl


def kernel(root_node_embeddings, trunk_node_inputs_indices, trunk_node_types, W1, b1, W2, b2):
    raise NotImplementedError("write your pallas kernel here")


# speedup vs baseline: 179.3139x; 179.3139x over previous
import functools

import jax
import jax.numpy as jnp
from jax.experimental import pallas as pl
from jax.experimental.pallas import tpu as pltpu

NUM_ROOT = 64
NUM_TRUNK = 10000
D = 128
IN_DEG = 2
BLK = 200
NUM_BLOCKS = NUM_TRUNK // BLK


def _dag_kernel(idx_ref, root_ref, w1a_ref, w1b_ref, b1_ref, w2_ref, b2_ref,
                out_ref, x0_ref, x1_ref, lev_ref):
    b = pl.program_id(0)
    s = b * BLK

    @pl.when(b == 0)
    def _init():
        out_ref[0:NUM_ROOT, :] = root_ref[...]
        out_ref[NUM_ROOT:, :] = jnp.zeros((NUM_TRUNK, D), jnp.float32)

    base = s + NUM_ROOT

    def lev_body(j, maxlev):
        p0 = idx_ref[2 * (s + j)]
        p1 = idx_ref[2 * (s + j) + 1]
        l0 = jnp.where(p0 >= base, lev_ref[jnp.maximum(p0 - base, 0)], 0)
        l1 = jnp.where(p1 >= base, lev_ref[jnp.maximum(p1 - base, 0)], 0)
        lv = jnp.maximum(l0, l1) + 1
        lev_ref[j] = lv
        return jnp.maximum(maxlev, lv)

    npass = jax.lax.fori_loop(0, BLK, lev_body, jnp.int32(0))

    def pass_body(_, carry):
        for j in range(BLK):
            i0 = idx_ref[2 * (s + j)]
            i1 = idx_ref[2 * (s + j) + 1]
            x0_ref[pl.ds(j, 1), :] = out_ref[pl.ds(i0, 1), :]
            x1_ref[pl.ds(j, 1), :] = out_ref[pl.ds(i1, 1), :]

        h = (jnp.dot(x0_ref[...], w1a_ref[...], preferred_element_type=jnp.float32)
             + jnp.dot(x1_ref[...], w1b_ref[...], preferred_element_type=jnp.float32))
        h = jax.nn.gelu(h + b1_ref[...])
        new = jnp.dot(h, w2_ref[...], preferred_element_type=jnp.float32)
        new = new + b2_ref[...]
        out_ref[pl.ds(base, BLK), :] = new
        return carry

    jax.lax.fori_loop(0, npass, pass_body, 0)


@jax.jit
def kernel(root_node_embeddings, trunk_node_inputs_indices, trunk_node_types,
           W1, b1, W2, b2):
    del trunk_node_types
    grid_spec = pltpu.PrefetchScalarGridSpec(
        num_scalar_prefetch=1,
        grid=(NUM_BLOCKS,),
        in_specs=[
            pl.BlockSpec((NUM_ROOT, D), lambda b, idx: (0, 0)),
            pl.BlockSpec((D, 2 * D), lambda b, idx: (0, 0)),
            pl.BlockSpec((D, 2 * D), lambda b, idx: (0, 0)),
            pl.BlockSpec((1, 2 * D), lambda b, idx: (0, 0)),
            pl.BlockSpec((2 * D, D), lambda b, idx: (0, 0)),
            pl.BlockSpec((1, D), lambda b, idx: (0, 0)),
        ],
        out_specs=pl.BlockSpec((NUM_ROOT + NUM_TRUNK, D), lambda b, idx: (0, 0)),
        scratch_shapes=[
            pltpu.VMEM((BLK, D), jnp.float32),
            pltpu.VMEM((BLK, D), jnp.float32),
            pltpu.SMEM((BLK,), jnp.int32),
        ],
    )
    out = pl.pallas_call(
        _dag_kernel,
        grid_spec=grid_spec,
        out_shape=jax.ShapeDtypeStruct((NUM_ROOT + NUM_TRUNK, D), jnp.float32),
        compiler_params=pltpu.CompilerParams(
            dimension_semantics=("arbitrary",),
        ),
    )(trunk_node_inputs_indices.reshape(-1),
      root_node_embeddings,
      W1[:D], W1[D:], b1.reshape(1, 2 * D), W2, b2.reshape(1, D))
    return out


# speedup vs baseline: 281.1776x; 1.5681x over previous
import jax
import jax.numpy as jnp
from jax.experimental import pallas as pl
from jax.experimental.pallas import tpu as pltpu

NUM_ROOT = 64
NUM_TRUNK = 10000
D = 128
IN_DEG = 2
BLK = 200
NUM_BLOCKS = NUM_TRUNK // BLK


def _dag_kernel(idx_ref, root_ref, w1a_ref, w1b_ref, b1_ref, w2_ref, b2_ref,
                out_ref, x0_ref, x1_ref):
    b = pl.program_id(0)
    s = b * BLK

    @pl.when(b == 0)
    def _init():
        out_ref[0:NUM_ROOT, :] = root_ref[...]
        out_ref[NUM_ROOT:, :] = jnp.zeros((NUM_TRUNK, D), jnp.float32)

    base = s + NUM_ROOT

    def pass_body(carry):
        p, _ = carry
        for j in range(BLK):
            i0 = idx_ref[2 * (s + j)]
            i1 = idx_ref[2 * (s + j) + 1]
            x0_ref[pl.ds(j, 1), :] = out_ref[pl.ds(i0, 1), :]
            x1_ref[pl.ds(j, 1), :] = out_ref[pl.ds(i1, 1), :]

        h = (jnp.dot(x0_ref[...], w1a_ref[...], preferred_element_type=jnp.float32)
             + jnp.dot(x1_ref[...], w1b_ref[...], preferred_element_type=jnp.float32))
        h = jax.nn.gelu(h + b1_ref[...])
        new = jnp.dot(h, w2_ref[...], preferred_element_type=jnp.float32)
        new = new + b2_ref[...]
        old = out_ref[pl.ds(base, BLK), :]
        nchanged = jnp.sum((new != old).astype(jnp.float32))
        out_ref[pl.ds(base, BLK), :] = new
        return (p + 1, nchanged > 0.0)

    def cond(carry):
        p, go = carry
        return jnp.logical_and(go, p < BLK)

    jax.lax.while_loop(cond, pass_body, (jnp.int32(0), jnp.bool_(True)))


@jax.jit
def kernel(root_node_embeddings, trunk_node_inputs_indices, trunk_node_types,
           W1, b1, W2, b2):
    del trunk_node_types
    grid_spec = pltpu.PrefetchScalarGridSpec(
        num_scalar_prefetch=1,
        grid=(NUM_BLOCKS,),
        in_specs=[
            pl.BlockSpec((NUM_ROOT, D), lambda b, idx: (0, 0)),
            pl.BlockSpec((D, 2 * D), lambda b, idx: (0, 0)),
            pl.BlockSpec((D, 2 * D), lambda b, idx: (0, 0)),
            pl.BlockSpec((1, 2 * D), lambda b, idx: (0, 0)),
            pl.BlockSpec((2 * D, D), lambda b, idx: (0, 0)),
            pl.BlockSpec((1, D), lambda b, idx: (0, 0)),
        ],
        out_specs=pl.BlockSpec((NUM_ROOT + NUM_TRUNK, D), lambda b, idx: (0, 0)),
        scratch_shapes=[
            pltpu.VMEM((BLK, D), jnp.float32),
            pltpu.VMEM((BLK, D), jnp.float32),
        ],
    )
    out = pl.pallas_call(
        _dag_kernel,
        grid_spec=grid_spec,
        out_shape=jax.ShapeDtypeStruct((NUM_ROOT + NUM_TRUNK, D), jnp.float32),
        compiler_params=pltpu.CompilerParams(
            dimension_semantics=("arbitrary",),
        ),
    )(trunk_node_inputs_indices.reshape(-1),
      root_node_embeddings,
      W1[:D], W1[D:], b1.reshape(1, 2 * D), W2, b2.reshape(1, D))
    return out
